# baseline (speedup 1.0000x reference)
import jax
import jax.numpy as jnp
from jax import lax
from jax.experimental import pallas as pl
from jax.experimental.pallas import tpu as pltpu
from jax.experimental.pallas import tpu_sc as plsc

D = 16
PACK = 8
CHUNK = 128


def _sc_geometry():
    try:
        info = plsc.get_sparse_core_info()
        return info.num_cores, info.num_subcores
    except Exception:
        return 2, 16


def _sc_gather(user_idx, item_idx, tab_u, tab_i, B):
    NC, NS = _sc_geometry()
    NW = NC * NS
    rows_per_w = B // NW
    n_chunks = rows_per_w // CHUNK
    mesh = plsc.VectorSubcoreMesh(core_axis_name="c", subcore_axis_name="s")

    def body(uidx_hbm, iidx_hbm, tabu_hbm, tabi_hbm,
             out_u, out_ib, out_i,
             idxu_v, idxi_v, divu_v, divi_v,
             bu0, bu1, bib0, bib1, bi0, bi1, sem0, sem1):
        wid = lax.axis_index("s") * NC + lax.axis_index("c")
        base = wid * rows_per_w
        pltpu.sync_copy(uidx_hbm.at[pl.ds(base, rows_per_w)], idxu_v)
        pltpu.sync_copy(iidx_hbm.at[pl.ds(base, rows_per_w)], idxi_v)
        for g in range(rows_per_w // 16):
            sl = pl.ds(g * 16, 16)
            divu_v[sl] = lax.shift_right_logical(idxu_v[sl], 3)
            divi_v[sl] = lax.shift_right_logical(idxi_v[sl], 3)

        bufs = [(bu0, bib0, bi0, sem0), (bu1, bib1, bi1, sem1)]

        def fire(j):
            bu, bib, bi, sem = bufs[j % 2]
            isl = pl.ds(j * CHUNK, CHUNK)
            return [
                pltpu.async_copy(tabu_hbm.at[divu_v.at[isl]], bu, sem),
                pltpu.async_copy(tabu_hbm.at[divi_v.at[isl]], bib, sem),
                pltpu.async_copy(tabi_hbm.at[divi_v.at[isl]], bi, sem),
            ]

        def drain_and_store(j, descs):
            for d in descs:
                d.wait()
            bu, bib, bi, _ = bufs[j % 2]
            osl = pl.ds(base + j * CHUNK, CHUNK)
            pltpu.sync_copy(bu, out_u.at[osl])
            pltpu.sync_copy(bib, out_ib.at[osl])
            pltpu.sync_copy(bi, out_i.at[osl])

        inflight = {}
        for j in range(n_chunks):
            if j >= 2:
                drain_and_store(j - 2, inflight.pop(j - 2))
            inflight[j] = fire(j)
        for j in sorted(inflight):
            drain_and_store(j, inflight[j])

    out_sds = jax.ShapeDtypeStruct((B, 128), jnp.float32)
    buf = pltpu.VMEM((CHUNK, 128), jnp.float32)
    k = pl.kernel(
        body,
        out_type=(out_sds, out_sds, out_sds),
        mesh=mesh,
        scratch_types=[
            pltpu.VMEM((rows_per_w,), jnp.int32),
            pltpu.VMEM((rows_per_w,), jnp.int32),
            pltpu.VMEM((rows_per_w,), jnp.int32),
            pltpu.VMEM((rows_per_w,), jnp.int32),
            buf, buf, buf, buf, buf, buf,
            pltpu.SemaphoreType.DMA,
            pltpu.SemaphoreType.DMA,
        ],
    )
    return k(user_idx, item_idx, tab_u, tab_i)


def _fold16(x):
    acc = x[:, 0:D]
    for s in range(1, PACK):
        acc = acc + x[:, s * D:(s + 1) * D]
    return acc


def _mlp_body(ru, rib, ri, selu, seli, w1t, b1, w2t, b2, w3t, b3,
              womf, womlp, bo, out):
    su = selu[...]
    si = seli[...]
    u = _fold16(ru[...] * su)
    ib = _fold16(rib[...] * si)
    it = _fold16(ri[...] * si)
    x = jnp.concatenate([u, it], axis=1)
    hp = jax.lax.Precision.HIGHEST
    h = jnp.maximum(jnp.dot(x, w1t[...], precision=hp) + b1[...], 0.0)
    h = jnp.maximum(jnp.dot(h, w2t[...], precision=hp) + b2[...], 0.0)
    h = jnp.maximum(jnp.dot(h, w3t[...], precision=hp) + b3[...], 0.0)
    mf = u * ib
    logit = (jnp.dot(mf, womf[...], precision=hp)
             + jnp.dot(h, womlp[...], precision=hp) + bo[...])
    out[...] = jax.nn.sigmoid(logit)


def _tc_mlp(ru, rib, ri, selu, seli, W1, b1, W2, b2, W3, b3, Wo, bo, B):
    blk = 2048
    grid = B // blk
    full = lambda shape: pl.BlockSpec(shape, lambda i: (0, 0))
    row = lambda: pl.BlockSpec((blk, 128), lambda i: (i, 0))
    return pl.pallas_call(
        _mlp_body,
        grid=(grid,),
        in_specs=[
            row(), row(), row(), row(), row(),
            full((32, 32)), full((1, 32)),
            full((32, 16)), full((1, 16)),
            full((16, 8)), full((1, 8)),
            full((16, 1)), full((8, 1)), full((1, 1)),
        ],
        out_specs=pl.BlockSpec((blk, 1), lambda i: (i, 0)),
        out_shape=jax.ShapeDtypeStruct((B, 1), jnp.float32),
    )(ru, rib, ri, selu, seli,
      W1.T, b1.reshape(1, 32),
      W2.T, b2.reshape(1, 16),
      W3.T, b3.reshape(1, 8),
      Wo[:, :D].T, Wo[:, D:].T, bo.reshape(1, 1))


def _selmat(idx, B):
    oh = jax.nn.one_hot(jnp.bitwise_and(idx, PACK - 1), PACK,
                        dtype=jnp.float32)
    return jnp.repeat(oh, D, axis=1)


def kernel(user_input, item_input, mf_user_table, mf_item_table,
           W1, b1, W2, b2, W3, b3, Wo, bo):
    B = user_input.shape[0]
    V = mf_user_table.shape[0]
    tab_u = mf_user_table.reshape(V // PACK, PACK * D)
    tab_i = mf_item_table.reshape(V // PACK, PACK * D)
    ru, rib, ri = _sc_gather(user_input, item_input, tab_u, tab_i, B)
    selu = _selmat(user_input, B)
    seli = _selmat(item_input, B)
    return _tc_mlp(ru, rib, ri, selu, seli,
                   W1, b1, W2, b2, W3, b3, Wo, bo, B)


# speedup vs baseline: 1.3176x; 1.3176x over previous
import jax
import jax.numpy as jnp
from jax import lax
from jax.experimental import pallas as pl
from jax.experimental.pallas import tpu as pltpu
from jax.experimental.pallas import tpu_sc as plsc

D = 16
CH = 32


def _sc_geometry():
    try:
        info = plsc.get_sparse_core_info()
        return info.num_cores, info.num_subcores
    except Exception:
        return 2, 16


def _sc_gather(user_idx, item_idx, tab_u, tab_i, B):
    NC, NS = _sc_geometry()
    NW = NC * NS
    rpw = B // NW
    n_ch = rpw // CH

    mesh = plsc.VectorSubcoreMesh(core_axis_name="c", subcore_axis_name="s")

    def body(uidx_hbm, iidx_hbm, tabu_hbm, tabi_hbm,
             out_u, out_ib, out_i,
             idxu_v, idxi_v, pb, ob, sem):
        wid = lax.axis_index("s") * NC + lax.axis_index("c")
        base = wid * rpw
        pltpu.sync_copy(uidx_hbm.at[pl.ds(base, rpw)], idxu_v)
        pltpu.sync_copy(iidx_hbm.at[pl.ds(base, rpw)], idxi_v)

        def run_stream(idx_v, tab, out):
            def wave(j, _):
                vecs = [idx_v[pl.ds(j * CH + g * D, D)] for g in range(CH // D)]
                for g, vec in enumerate(vecs):
                    for l in range(D):
                        v = vec[l]
                        row8 = pl.multiple_of(
                            lax.shift_left(lax.shift_right_logical(v, 3), 3), 8)
                        slot = (g * D + l) * 8
                        pltpu.async_copy(tab.at[pl.ds(row8, 8), :],
                                         pb.at[pl.ds(slot, 8), :], sem)
                pltpu.make_async_copy(tab.at[pl.ds(0, CH * 8), :], pb,
                                      sem).wait()
                for g, vec in enumerate(vecs):
                    for l in range(D):
                        r = vec[l] & 7
                        val = pb[(g * D + l) * 8 + r, :]
                        ob[g * D + l, :] = val
                pltpu.sync_copy(ob, out.at[pl.ds(base + j * CH, CH)])
                return 0

            lax.fori_loop(0, n_ch, wave, 0)

        run_stream(idxu_v, tabu_hbm, out_u)
        run_stream(idxi_v, tabu_hbm, out_ib)
        run_stream(idxi_v, tabi_hbm, out_i)

    out_sds = jax.ShapeDtypeStruct((B, D), jnp.float32)
    k = pl.kernel(
        body,
        out_type=(out_sds, out_sds, out_sds),
        mesh=mesh,
        scratch_types=[
            pltpu.VMEM((rpw,), jnp.int32),
            pltpu.VMEM((rpw,), jnp.int32),
            pltpu.VMEM((CH * 8, D), jnp.float32),
            pltpu.VMEM((CH, D), jnp.float32),
            pltpu.SemaphoreType.DMA,
        ],
    )
    return k(user_idx, item_idx, tab_u, tab_i)


def _mlp_body(ru, rib, ri, w1t, b1, w2t, b2, w3t, b3, womf, womlp, bo, out):
    u = ru[...]
    x = jnp.concatenate([u, ri[...]], axis=1)
    hp = jax.lax.Precision.HIGHEST
    h = jnp.maximum(jnp.dot(x, w1t[...], precision=hp) + b1[...], 0.0)
    h = jnp.maximum(jnp.dot(h, w2t[...], precision=hp) + b2[...], 0.0)
    h = jnp.maximum(jnp.dot(h, w3t[...], precision=hp) + b3[...], 0.0)
    mf = u * rib[...]
    logit = (jnp.dot(mf, womf[...], precision=hp)
             + jnp.dot(h, womlp[...], precision=hp) + bo[...])
    out[...] = jax.nn.sigmoid(logit)


def _tc_mlp(ru, rib, ri, W1, b1, W2, b2, W3, b3, Wo, bo, B):
    blk = 4096
    grid = B // blk
    full = lambda shape: pl.BlockSpec(shape, lambda i: (0, 0))
    row = lambda: pl.BlockSpec((blk, D), lambda i: (i, 0))
    return pl.pallas_call(
        _mlp_body,
        grid=(grid,),
        in_specs=[
            row(), row(), row(),
            full((32, 32)), full((1, 32)),
            full((32, 16)), full((1, 16)),
            full((16, 8)), full((1, 8)),
            full((16, 1)), full((8, 1)), full((1, 1)),
        ],
        out_specs=pl.BlockSpec((blk, 1), lambda i: (i, 0)),
        out_shape=jax.ShapeDtypeStruct((B, 1), jnp.float32),
    )(ru, rib, ri,
      W1.T, b1.reshape(1, 32),
      W2.T, b2.reshape(1, 16),
      W3.T, b3.reshape(1, 8),
      Wo[:, :D].T, Wo[:, D:].T, bo.reshape(1, 1))


def kernel(user_input, item_input, mf_user_table, mf_item_table,
           W1, b1, W2, b2, W3, b3, Wo, bo):
    B = user_input.shape[0]
    ru, rib, ri = _sc_gather(user_input, item_input,
                             mf_user_table, mf_item_table, B)
    return _tc_mlp(ru, rib, ri, W1, b1, W2, b2, W3, b3, Wo, bo, B)
